# baseline (speedup 1.0000x reference)
import jax
import jax.numpy as jnp
from jax.experimental import pallas as pl
from jax.experimental.pallas import tpu as pltpu

_B, _L, _C = 4096, 20, 1000
_N = _B * _L
_ROWS = 1024


def _onehot_block(x_ref, o_ref):
    xb = x_ref[...]
    act = xb[:, 0:1].astype(jnp.int32)
    dur = xb[:, 1:2]
    col = jax.lax.broadcasted_iota(jnp.int32, (_ROWS, _C + 1), 1)
    o_ref[...] = (col == act).astype(jnp.float32)
    o_ref[:, _C:_C + 1] = dur


def kernel(x):
    xf = x.reshape(_N, 2)
    out = pl.pallas_call(
        _onehot_block,
        grid=(_N // _ROWS,),
        in_specs=[pl.BlockSpec((_ROWS, 2), lambda i: (i, 0))],
        out_specs=pl.BlockSpec((_ROWS, _C + 1), lambda i: (i, 0)),
        out_shape=jax.ShapeDtypeStruct((_N, _C + 1), jnp.float32),
        compiler_params=pltpu.CompilerParams(
            dimension_semantics=("parallel",),
        ),
    )(xf)
    return out.reshape(_B, _L, _C + 1)


# speedup vs baseline: 1.0629x; 1.0629x over previous
import jax
import jax.numpy as jnp
from jax.experimental import pallas as pl
from jax.experimental.pallas import tpu as pltpu

_B, _L, _C = 4096, 20, 1000
_N = _B * _L
_ROWS = 1024
_NSTEP = _N // _ROWS
_NBUF = 8


def _onehot_multibuf(x_ref, o_ref, buf, sems):
    col = jax.lax.broadcasted_iota(jnp.int32, (_ROWS, _C + 1), 1)

    def step(i, carry):
        slot = jax.lax.rem(i, _NBUF)

        @pl.when(i >= _NBUF)
        def _wait_prev():
            prev = i - _NBUF
            pltpu.make_async_copy(
                buf.at[slot],
                o_ref.at[pl.ds(prev * _ROWS, _ROWS), :],
                sems.at[slot],
            ).wait()

        xb = x_ref[:, pl.ds(i * _ROWS, _ROWS)]
        xt = jax.lax.transpose(xb, (1, 0))
        act = xt[:, 0:1].astype(jnp.int32)
        dur = xt[:, 1:2]
        buf[slot] = (col == act).astype(jnp.float32)
        buf[slot, :, _C:_C + 1] = dur
        pltpu.make_async_copy(
            buf.at[slot],
            o_ref.at[pl.ds(i * _ROWS, _ROWS), :],
            sems.at[slot],
        ).start()
        return carry

    jax.lax.fori_loop(0, _NSTEP, step, 0)

    def drain(i, carry):
        slot = jax.lax.rem(i, _NBUF)
        pltpu.make_async_copy(
            buf.at[slot],
            o_ref.at[pl.ds(i * _ROWS, _ROWS), :],
            sems.at[slot],
        ).wait()
        return carry

    jax.lax.fori_loop(_NSTEP - _NBUF, _NSTEP, drain, 0)


def kernel(x):
    xt = x.reshape(_N, 2).T
    out = pl.pallas_call(
        _onehot_multibuf,
        in_specs=[pl.BlockSpec(memory_space=pltpu.VMEM)],
        out_specs=pl.BlockSpec(memory_space=pl.ANY),
        out_shape=jax.ShapeDtypeStruct((_N, _C + 1), jnp.float32),
        scratch_shapes=[
            pltpu.VMEM((_NBUF, _ROWS, _C + 1), jnp.float32),
            pltpu.SemaphoreType.DMA((_NBUF,)),
        ],
    )(xt)
    return out.reshape(_B, _L, _C + 1)
